# Initial kernel scaffold; baseline (speedup 1.0000x reference)
import jax
import jax.numpy as jnp
from jax.experimental import pallas as pl


def kernel(x_machine, x_operation, ei_assignment, ei_completion, ei_type_valid, ei_logical, ea_assignment, ea_completion, W_me, b_me, W_oe, b_oe, W_ea, b_ea, W_ga, b_ga, W_ec, b_ec, W_gc, b_gc, W_ll, b_ll, W_lr, W_tl, b_tl, W_tr):
    raise NotImplementedError("write your pallas kernel here")


# speedup vs baseline: 5.4161x; 5.4161x over previous
import functools

import jax
import jax.numpy as jnp
from jax import lax
from jax.experimental import pallas as pl
from jax.experimental.pallas import tpu as pltpu
from jax.experimental.pallas import tpu_sc as plsc

NM = 5000
NO = 50000
NE = 800000
H = 64

NC = 2
NS = 16
LN = 16

EPAD = 802816
NPAD = EPAD - NE
EW = EPAD // 32
ET = EPAD // 16
NL_TAB = 50176
NT_TAB = 5120
KAC = 128
KLT = 1024


def _encode_m_body(xma_r, wme_r, bme_r, xm_out):
    xma, wme, bme = xma_r[...], wme_r[...], bme_r[...]
    accm = jnp.broadcast_to(bme, (NM, H))
    for k in range(3):
        accm = accm + xma[:, k:k + 1] * wme[k]
    xm_out[...] = accm


ENC_BLK = 5000


def _encode_o_body(xop_r, woe_r, boe_r, xo_out):
    xop, woe, boe = xop_r[...], woe_r[...], boe_r[...]
    acco = jnp.broadcast_to(boe, (ENC_BLK, H))
    for k in range(4):
        acco = acco + xop[:, k:k + 1] * woe[k]
    xo_out[...] = acco


def _sc_body(xm_hbm, xo_hbm, srcA, ea0, ea1, ea2, srcC, eaC, srcL, dstL, srcT, dstT,
             wea_hbm, bea_hbm, wec_hbm, bec_hbm,
             saP, scP, qL_out, qT_out, stage_hbm, w_hbm,
             cnt_tab, q_tab, src_buf, dst_buf, idx_buf, rows, a_buf,
             w_buf, b_buf, wc_buf, bc_buf, acc_buf, red_buf, red_acc, sem):
    cid = lax.axis_index("c")
    sid = lax.axis_index("s")
    wid = sid * NC + cid

    pltpu.sync_copy(wea_hbm, w_buf)
    pltpu.sync_copy(bea_hbm, b_buf)
    pltpu.sync_copy(wec_hbm, wc_buf)
    pltpu.sync_copy(bec_hbm, bc_buf)

    ones = jnp.ones((LN,), jnp.float32)

    def zero_ref(ref, n):
        def body(i, _):
            ref[pl.ds(i * LN, LN)] = jnp.zeros((LN,), jnp.float32)
            return 0
        lax.fori_loop(0, n // LN, body, 0)

    def sage_phase(src_hbm, dst_hbm, tab_n, q_out):
        rn = tab_n // NS
        sbase = (cid * NS + sid) * NL_TAB
        wbase = cid * NL_TAB
        zero_ref(cnt_tab, tab_n)
        def hist_round(r, _):
            base = sid * ET + r * KLT
            pltpu.sync_copy(dst_hbm.at[pl.ds(base, KLT)], dst_buf)
            def inner(i, _):
                d = dst_buf[pl.ds(i * LN, LN)]
                plsc.addupdate_scatter(cnt_tab, [d], ones)
                return 0
            lax.fori_loop(0, KLT // LN, inner, 0)
            return 0
        lax.fori_loop(0, ET // KLT, hist_round, 0)
        pltpu.sync_copy(cnt_tab.at[pl.ds(0, tab_n)],
                        stage_hbm.at[pl.ds(sbase, tab_n)])
        plsc.subcore_barrier()
        zero_ref(red_acc, rn)
        def red_t(t, _):
            pltpu.sync_copy(
                stage_hbm.at[pl.ds((cid * NS + t) * NL_TAB + sid * rn, rn)],
                red_buf.at[pl.ds(0, rn)])
            def addv(i, _):
                red_acc[pl.ds(i * LN, LN)] = (red_acc[pl.ds(i * LN, LN)]
                                              + red_buf[pl.ds(i * LN, LN)])
                return 0
            lax.fori_loop(0, rn // LN, addv, 0)
            return 0
        lax.fori_loop(0, NS, red_t, 0)
        def wv(i, _):
            c = red_acc[pl.ds(i * LN, LN)]
            red_acc[pl.ds(i * LN, LN)] = 1.0 / jnp.maximum(c, 1.0)
            return 0
        lax.fori_loop(0, rn // LN, wv, 0)
        pltpu.sync_copy(red_acc.at[pl.ds(0, rn)],
                        w_hbm.at[pl.ds(wbase + sid * rn, rn)])
        plsc.subcore_barrier()
        pltpu.sync_copy(w_hbm.at[pl.ds(wbase, tab_n)],
                        cnt_tab.at[pl.ds(0, tab_n)])
        zero_ref(q_tab, tab_n)
        def q_round(r, _):
            base = sid * ET + r * KLT
            pltpu.sync_copy(dst_hbm.at[pl.ds(base, KLT)], dst_buf)
            pltpu.sync_copy(src_hbm.at[pl.ds(base, KLT)], src_buf)
            def inner(i, _):
                d = dst_buf[pl.ds(i * LN, LN)]
                s = src_buf[pl.ds(i * LN, LN)]
                w = plsc.load_gather(cnt_tab, [d])
                plsc.addupdate_scatter(q_tab, [s], w)
                return 0
            lax.fori_loop(0, KLT // LN, inner, 0)
            return 0
        lax.fori_loop(0, ET // KLT, q_round, 0)
        pltpu.sync_copy(q_tab.at[pl.ds(0, tab_n)],
                        stage_hbm.at[pl.ds(sbase, tab_n)])
        plsc.subcore_barrier()
        zero_ref(red_acc, rn)
        lax.fori_loop(0, NS, red_t, 0)
        pltpu.sync_copy(red_acc.at[pl.ds(0, rn)], q_out.at[pl.ds(sid * rn, rn)])
        plsc.subcore_barrier()

    @pl.when(cid == 0)
    def _():
        sage_phase(srcL, dstL, NL_TAB, qL_out)

    @pl.when(cid == 1)
    def _():
        sage_phase(srcT, dstT, NT_TAB, qT_out)

    def gine_phase(tab_hbm, src_hbm, attr_rows, wvecs, bvecs, out_hbm):
        def rnd(r, accs):
            base = wid * EW + r * KAC
            pltpu.sync_copy(src_hbm.at[pl.ds(base, KAC)], idx_buf)
            for k in range(len(attr_rows)):
                pltpu.sync_copy(attr_rows[k].at[pl.ds(base, KAC)],
                                a_buf.at[pl.ds(k * KAC, KAC)])
            pltpu.async_copy(tab_hbm.at[idx_buf], rows, sem).wait()
            nk = len(attr_rows)
            def group(g, accs):
                av = [a_buf[pl.ds(k * KAC + g * LN, LN)] for k in range(nk)]
                accs = list(accs)
                for j in range(LN):
                    e = g * LN + j
                    a = [av[k][j] for k in range(nk)]
                    for c in range(4):
                        ev = bvecs[c]
                        for k in range(nk):
                            ev = ev + a[k] * wvecs[k][c]
                        rv = rows[e, pl.ds(c * LN, LN)]
                        accs[c] = accs[c] + jnp.maximum(rv + ev, 0.0)
                return tuple(accs)
            return lax.fori_loop(0, KAC // LN, group, accs)
        z = jnp.zeros((LN,), jnp.float32)
        accs = lax.fori_loop(0, EW // KAC, rnd, (z, z, z, z))
        for c in range(4):
            acc_buf[pl.ds(c * LN, LN)] = accs[c]
        pltpu.sync_copy(acc_buf, out_hbm.at[wid])

    wA = [[w_buf[k, pl.ds(c * LN, LN)] for c in range(4)] for k in range(3)]
    bA = [b_buf[pl.ds(c * LN, LN)] for c in range(4)]
    wC = [[wc_buf[pl.ds(c * LN, LN)] for c in range(4)]]
    bC = [bc_buf[pl.ds(c * LN, LN)] for c in range(4)]
    gine_phase(xm_hbm, srcA, [ea0, ea1, ea2], wA, bA, saP)
    gine_phase(xo_hbm, srcC, [eaC], wC, bC, scP)


def _combine_body(*refs):
    (saP, scP, qL, qT, xmT, xoT,
     wme, bme, woe, boe, bea, bec,
     wga, bga, wgc, bgc, wll, bll, wlr, wtl, btl, wtr) = (
        r[...] for r in refs[:-1])
    out_ref = refs[-1]

    def vdot(v, w):
        return jnp.sum(v[:, None] * w, axis=0)

    s_a = jnp.sum(saP, axis=0)
    s_c = jnp.sum(scP, axis=0)
    xm0 = vdot(xmT[:, 0], wme) + bme
    s_a = s_a - NPAD * jnp.maximum(xm0 + bea, 0.0)
    xo0 = vdot(xoT[:, 0], woe) + boe
    s_c = s_c - NPAD * jnp.maximum(xo0 + bec, 0.0)
    qLr = qL[:NO]
    qTr = qT[:NM]
    r4 = jnp.sum(qLr[None, :] * xoT, axis=1)
    s_l = vdot(r4, woe) + jnp.sum(qLr) * boe
    t4 = jnp.sum(qTr[None, :] * xoT[:, :NM], axis=1)
    s_t = vdot(t4, woe) + jnp.sum(qTr) * boe
    mean_xo = vdot(jnp.mean(xoT, axis=1), woe) + boe
    mean_xm = vdot(jnp.mean(xmT, axis=1), wme) + bme
    ga = vdot(mean_xo + s_a / NO, wga) + bga
    gc = vdot(mean_xo + s_c / NO, wgc) + bgc
    gl = vdot(s_l / NO, wll) + bll + vdot(mean_xo, wlr)
    g_o = (ga + gc + gl) / 3.0
    g_m = vdot(s_t / NM, wtl) + btl + vdot(mean_xm, wtr)
    out_ref[...] = jnp.concatenate([g_m, g_o]).reshape(1, 2 * H)


def kernel(x_machine, x_operation, ei_assignment, ei_completion, ei_type_valid,
           ei_logical, ea_assignment, ea_completion, W_me, b_me, W_oe, b_oe,
           W_ea, b_ea, W_ga, b_ga, W_ec, b_ec, W_gc, b_gc, W_ll, b_ll, W_lr,
           W_tl, b_tl, W_tr):
    f32 = jnp.float32
    xm = pl.pallas_call(
        _encode_m_body,
        out_shape=jax.ShapeDtypeStruct((NM, H), f32),
    )(x_machine, W_me, b_me)
    xo = pl.pallas_call(
        _encode_o_body,
        grid=(NO // ENC_BLK,),
        in_specs=[pl.BlockSpec((ENC_BLK, 4), lambda i: (i, 0)),
                  pl.BlockSpec((4, H), lambda i: (0, 0)),
                  pl.BlockSpec((H,), lambda i: (0,))],
        out_specs=pl.BlockSpec((ENC_BLK, H), lambda i: (i, 0)),
        out_shape=jax.ShapeDtypeStruct((NO, H), f32),
    )(x_operation, W_oe, b_oe)

    pe = EPAD - NE
    src_a = jnp.pad(ei_assignment[0], (0, pe))
    ea_p = jnp.pad(ea_assignment, ((0, pe), (0, 0)))
    src_c = jnp.pad(ei_completion[0], (0, pe))
    ea_c = jnp.pad(ea_completion[:, 0], (0, pe))
    src_l = jnp.pad(ei_logical[0], (0, pe), constant_values=NO)
    dst_l = jnp.pad(ei_logical[1], (0, pe), constant_values=NO)
    src_t = jnp.pad(ei_type_valid[0], (0, pe), constant_values=NM)
    dst_t = jnp.pad(ei_type_valid[1], (0, pe), constant_values=NM)

    mesh = plsc.VectorSubcoreMesh(core_axis_name="c", subcore_axis_name="s")
    sc = pl.kernel(
        _sc_body,
        out_type=(jax.ShapeDtypeStruct((32, H), f32),
                  jax.ShapeDtypeStruct((32, H), f32),
                  jax.ShapeDtypeStruct((NL_TAB,), f32),
                  jax.ShapeDtypeStruct((NT_TAB,), f32),
                  jax.ShapeDtypeStruct((2 * NS * NL_TAB,), f32),
                  jax.ShapeDtypeStruct((2 * NL_TAB,), f32)),
        mesh=mesh,
        compiler_params=pltpu.CompilerParams(use_tc_tiling_on_sc=False, needs_layout_passes=False),
        scratch_types=[
            pltpu.VMEM((NL_TAB,), f32),
            pltpu.VMEM((NL_TAB,), f32),
            pltpu.VMEM((KLT,), jnp.int32),
            pltpu.VMEM((KLT,), jnp.int32),
            pltpu.VMEM((KAC,), jnp.int32),
            pltpu.VMEM((KAC, H), f32),
            pltpu.VMEM((3 * KAC,), f32),
            pltpu.VMEM((3, H), f32),
            pltpu.VMEM((H,), f32),
            pltpu.VMEM((H,), f32),
            pltpu.VMEM((H,), f32),
            pltpu.VMEM((H,), f32),
            pltpu.VMEM((NL_TAB // NS,), f32),
            pltpu.VMEM((NL_TAB // NS,), f32),
            pltpu.SemaphoreType.DMA,
        ],
    )
    saP, scP, qL, qT, _, _ = sc(xm, xo, src_a, ea_p[:, 0], ea_p[:, 1], ea_p[:, 2],
                          src_c, ea_c, src_l, dst_l, src_t, dst_t,
                          W_ea, b_ea, W_ec[0], b_ec)

    out = pl.pallas_call(
        _combine_body,
        out_shape=jax.ShapeDtypeStruct((1, 2 * H), f32),
    )(saP, scP, qL, qT, x_machine.T, x_operation.T,
      W_me, b_me, W_oe, b_oe, b_ea, b_ec,
      W_ga, b_ga, W_gc, b_gc, W_ll, b_ll, W_lr, W_tl, b_tl, W_tr)
    return out


# speedup vs baseline: 7.1719x; 1.3242x over previous
import jax
import jax.numpy as jnp
from jax import lax
from jax.experimental import pallas as pl
from jax.experimental.pallas import tpu as pltpu
from jax.experimental.pallas import tpu_sc as plsc

NM = 5000
NO = 50000
NE = 800000
H = 64

NC = 2
NS = 16
LN = 16

EPAD = 802816
NPAD = EPAD - NE
EW = EPAD // 32
ET = EPAD // 16
NL_TAB = 50176
NT_TAB = 5120
KG = 224
RG = EW // KG
KS = 3136
RS = ET // KS


def _encode_m_body(xma_r, wme_r, bme_r, xm_out):
    xma, wme, bme = xma_r[...], wme_r[...], bme_r[...]
    accm = jnp.broadcast_to(bme, (NM, H))
    for k in range(3):
        accm = accm + xma[:, k:k + 1] * wme[k]
    xm_out[...] = accm


ENC_BLK = 5000


def _encode_o_body(xop_r, woe_r, boe_r, xo_out):
    xop, woe, boe = xop_r[...], woe_r[...], boe_r[...]
    acco = jnp.broadcast_to(boe, (ENC_BLK, H))
    for k in range(4):
        acco = acco + xop[:, k:k + 1] * woe[k]
    xo_out[...] = acco


def _sc_body(xm_hbm, xo_hbm, strA, strC, strL, strT,
             wea_hbm, bea_hbm, wec_hbm, bec_hbm,
             saP, scP, qL_out, qT_out, stage_hbm, w_hbm,
             cnt_tab, q_tab, lt0, lt1, rows0, rows1,
             w_buf, b_buf, wc_buf, bc_buf, acc_buf,
             sem_i0, sem_i1, sem_g0, sem_g1):
    f32 = jnp.float32
    cid = lax.axis_index("c")
    sid = lax.axis_index("s")
    wid = sid * NC + cid
    lt_bufs = (lt0, lt1)
    ia_bufs = (lt0, lt1)
    row_bufs = (rows0, rows1)
    sem_ia = (sem_i0, sem_i1)
    sem_g = (sem_g0, sem_g1)

    pltpu.sync_copy(wea_hbm, w_buf)
    pltpu.sync_copy(bea_hbm, b_buf)
    pltpu.sync_copy(wec_hbm, wc_buf)
    pltpu.sync_copy(bec_hbm, bc_buf)

    ones = jnp.ones((LN,), f32)

    def zero_ref(ref, n):
        def body(i, _):
            for k in range(4):
                ref[pl.ds(i * 4 * LN + k * LN, LN)] = jnp.zeros((LN,), f32)
            return 0
        lax.fori_loop(0, n // (4 * LN), body, 0)

    def sage_phase(str_hbm, tab_n, nh, q_out):
        half = tab_n // nh
        rn_h = half // NS
        rn = tab_n // NS
        sbase = (cid * NS + sid) * NL_TAB
        wbase = cid * NL_TAB
        tbase = sid * (2 * ET)

        def lt_desc(r, b, n, make):
            mk = pltpu.make_async_copy if make else pltpu.async_copy
            return mk(str_hbm.at[pl.ds(tbase + r * (2 * KS), n * KS)],
                      lt_bufs[b].at[pl.ds(0, n * KS)], sem_ia[b])

        def pass_over_edges(nwords, process):
            lt_desc(0, 0, nwords, False)
            def pair(rr, _):
                for b in range(2):
                    r = 2 * rr + b
                    lt_desc(r, b, nwords, True).wait()
                    @pl.when(r < RS - 1)
                    def _():
                        lt_desc(r + 1, 1 - b, nwords, False)
                    process(lt_bufs[b])
                return 0
            lax.fori_loop(0, RS // 2, pair, 0)

        def red_desc(t, dst, n, make):
            mk = pltpu.make_async_copy if make else pltpu.async_copy
            return mk(
                stage_hbm.at[pl.ds((cid * NS + t) * NL_TAB + sid * n, n)],
                dst.at[pl.ds(t * n, n)], sem_g0)

        for ph in range(nh):
            lo = ph * half
            zero_ref(cnt_tab, half)
            def hist(buf):
                def inner(i, _):
                    d = buf[pl.ds(i * LN, LN)]
                    if nh == 1:
                        plsc.addupdate_scatter(cnt_tab, [d], ones)
                    else:
                        m = (d >= lo) & (d < lo + half)
                        plsc.addupdate_scatter(cnt_tab, [d - lo], ones,
                                               mask=m)
                    return 0
                lax.fori_loop(0, KS // LN, inner, 0)
            pass_over_edges(1, hist)
            pltpu.sync_copy(cnt_tab.at[pl.ds(0, half)],
                            stage_hbm.at[pl.ds(sbase, half)])
            plsc.subcore_barrier()
            for t in range(NS):
                red_desc(t, cnt_tab, rn_h, False)
            for t in range(NS):
                red_desc(t, cnt_tab, rn_h, True).wait()
            def red_sum(i, _):
                acc = cnt_tab[pl.ds(i * LN, LN)]
                for t in range(1, NS):
                    acc = acc + cnt_tab[pl.ds(t * rn_h + i * LN, LN)]
                cnt_tab[pl.ds(i * LN, LN)] = 1.0 / jnp.maximum(acc, 1.0)
                return 0
            lax.fori_loop(0, rn_h // LN, red_sum, 0)
            pltpu.sync_copy(
                cnt_tab.at[pl.ds(0, rn_h)],
                w_hbm.at[pl.ds(wbase + lo + sid * rn_h, rn_h)])
            plsc.subcore_barrier()

        zero_ref(q_tab, tab_n)
        for ph in range(nh):
            lo = ph * half
            pltpu.sync_copy(w_hbm.at[pl.ds(wbase + lo, half)],
                            cnt_tab.at[pl.ds(0, half)])
            def qpass(buf):
                def inner(i, _):
                    d = buf[pl.ds(i * LN, LN)]
                    sv = buf[pl.ds(KS + i * LN, LN)]
                    if nh == 1:
                        w = plsc.load_gather(cnt_tab, [d])
                        plsc.addupdate_scatter(q_tab, [sv], w)
                    else:
                        m = (d >= lo) & (d < lo + half)
                        w = plsc.load_gather(cnt_tab, [d - lo], mask=m)
                        plsc.addupdate_scatter(q_tab, [sv], w, mask=m)
                    return 0
                lax.fori_loop(0, KS // LN, inner, 0)
            pass_over_edges(2, qpass)
        pltpu.sync_copy(q_tab.at[pl.ds(0, tab_n)],
                        stage_hbm.at[pl.ds(sbase, tab_n)])
        plsc.subcore_barrier()
        for t in range(NS):
            red_desc(t, q_tab, rn, False)
        for t in range(NS):
            red_desc(t, q_tab, rn, True).wait()
        def qred_sum(i, _):
            acc = q_tab[pl.ds(i * LN, LN)]
            for t in range(1, NS):
                acc = acc + q_tab[pl.ds(t * rn + i * LN, LN)]
            q_tab[pl.ds(i * LN, LN)] = acc
            return 0
        lax.fori_loop(0, rn // LN, qred_sum, 0)
        pltpu.sync_copy(q_tab.at[pl.ds(0, rn)], q_out.at[pl.ds(sid * rn, rn)])

    @pl.when(cid == 0)
    def _():
        sage_phase(strL, NL_TAB, 2, qL_out)

    @pl.when(cid == 1)
    def _():
        sage_phase(strT, NT_TAB, 1, qT_out)

    def gine_phase(tab_hbm, str_hbm, nk, wvecs, bvecs, out_hbm):
        bs = (1 + nk) * KG
        gbase = wid * RG

        def ia_desc(b, make):
            mk = pltpu.make_async_copy if make else pltpu.async_copy
            return mk(str_hbm.at[pl.ds((gbase + 0) * bs, bs)],
                      ia_bufs[b].at[pl.ds(0, bs)], sem_ia[b])

        def fire_ia(r, b):
            pltpu.async_copy(str_hbm.at[pl.ds((gbase + r) * bs, bs)],
                             ia_bufs[b].at[pl.ds(0, bs)], sem_ia[b])

        def wait_ia(b):
            ia_desc(b, True).wait()

        def g_desc(b, make):
            mk = pltpu.make_async_copy if make else pltpu.async_copy
            return mk(tab_hbm.at[ia_bufs[b].at[pl.ds(0, KG)]], row_bufs[b],
                      sem_g[b])

        def fire_gather(b):
            g_desc(b, False)

        def wait_gather(b):
            g_desc(b, True).wait()

        def compute(b, accs):
            buf = ia_bufs[b]
            rows = row_bufs[b]
            def group(g, accs):
                av = [plsc.bitcast(
                    buf[pl.ds((1 + k) * KG + g * LN, LN)], f32)
                    for k in range(nk)]
                accs = list(accs)
                for j in range(LN):
                    e = g * LN + j
                    a = [av[k][j] for k in range(nk)]
                    for c in range(4):
                        ev = bvecs[c]
                        for k in range(nk):
                            ev = ev + a[k] * wvecs[k][c]
                        rv = rows[e, pl.ds(c * LN, LN)]
                        accs[c] = accs[c] + jnp.maximum(rv + ev, 0.0)
                return tuple(accs)
            return lax.fori_loop(0, KG // LN, group, accs)

        fire_ia(0, 0)
        wait_ia(0)
        fire_gather(0)
        fire_ia(1, 1)

        def pair(rr, accs):
            for b in range(2):
                r = 2 * rr + b
                wait_gather(b)
                @pl.when(r < RG - 1)
                def _():
                    wait_ia(1 - b)
                    fire_gather(1 - b)
                accs = compute(b, accs)
                @pl.when(r < RG - 2)
                def _():
                    fire_ia(r + 2, b)
            return accs
        z = jnp.zeros((LN,), f32)
        accs = lax.fori_loop(0, RG // 2, pair, (z, z, z, z))
        for c in range(4):
            acc_buf[pl.ds(c * LN, LN)] = accs[c]
        pltpu.sync_copy(acc_buf, out_hbm.at[wid])

    wA = [[w_buf[k, pl.ds(c * LN, LN)] for c in range(4)] for k in range(3)]
    bA = [b_buf[pl.ds(c * LN, LN)] for c in range(4)]
    wC = [[wc_buf[pl.ds(c * LN, LN)] for c in range(4)]]
    bC = [bc_buf[pl.ds(c * LN, LN)] for c in range(4)]
    gine_phase(xm_hbm, strA, 3, wA, bA, saP)
    gine_phase(xo_hbm, strC, 1, wC, bC, scP)


def _combine_body(*refs):
    (saP, scP, qL, qT, xmT, xoT,
     wme, bme, woe, boe, bea, bec,
     wga, bga, wgc, bgc, wll, bll, wlr, wtl, btl, wtr) = (
        r[...] for r in refs[:-1])
    out_ref = refs[-1]

    def vdot(v, w):
        return jnp.sum(v[:, None] * w, axis=0)

    s_a = jnp.sum(saP, axis=0)
    s_c = jnp.sum(scP, axis=0)
    xm0 = vdot(xmT[:, 0], wme) + bme
    s_a = s_a - NPAD * jnp.maximum(xm0 + bea, 0.0)
    xo0 = vdot(xoT[:, 0], woe) + boe
    s_c = s_c - NPAD * jnp.maximum(xo0 + bec, 0.0)
    qLr = qL[:NO]
    qTr = qT[:NM]
    r4 = jnp.sum(qLr[None, :] * xoT, axis=1)
    s_l = vdot(r4, woe) + jnp.sum(qLr) * boe
    t4 = jnp.sum(qTr[None, :] * xoT[:, :NM], axis=1)
    s_t = vdot(t4, woe) + jnp.sum(qTr) * boe
    mean_xo = vdot(jnp.mean(xoT, axis=1), woe) + boe
    mean_xm = vdot(jnp.mean(xmT, axis=1), wme) + bme
    ga = vdot(mean_xo + s_a / NO, wga) + bga
    gc = vdot(mean_xo + s_c / NO, wgc) + bgc
    gl = vdot(s_l / NO, wll) + bll + vdot(mean_xo, wlr)
    g_o = (ga + gc + gl) / 3.0
    g_m = vdot(s_t / NM, wtl) + btl + vdot(mean_xm, wtr)
    out_ref[...] = jnp.concatenate([g_m, g_o]).reshape(1, 2 * H)


def _pack_stream(cols, nrows, blk):
    arr = jnp.stack(cols, axis=0)
    arr = arr.reshape(nrows, EPAD // blk, blk)
    return arr.transpose(1, 0, 2).reshape(-1)


def kernel(x_machine, x_operation, ei_assignment, ei_completion, ei_type_valid,
           ei_logical, ea_assignment, ea_completion, W_me, b_me, W_oe, b_oe,
           W_ea, b_ea, W_ga, b_ga, W_ec, b_ec, W_gc, b_gc, W_ll, b_ll, W_lr,
           W_tl, b_tl, W_tr):
    f32 = jnp.float32
    i32 = jnp.int32
    xm = pl.pallas_call(
        _encode_m_body,
        out_shape=jax.ShapeDtypeStruct((NM, H), f32),
    )(x_machine, W_me, b_me)
    xo = pl.pallas_call(
        _encode_o_body,
        grid=(NO // ENC_BLK,),
        in_specs=[pl.BlockSpec((ENC_BLK, 4), lambda i: (i, 0)),
                  pl.BlockSpec((4, H), lambda i: (0, 0)),
                  pl.BlockSpec((H,), lambda i: (0,))],
        out_specs=pl.BlockSpec((ENC_BLK, H), lambda i: (i, 0)),
        out_shape=jax.ShapeDtypeStruct((NO, H), f32),
    )(x_operation, W_oe, b_oe)

    pe = EPAD - NE
    def pad(x, v=0):
        return jnp.pad(x, (0, pe), constant_values=v)
    bc = lambda x: lax.bitcast_convert_type(x, i32)
    str_a = _pack_stream(
        [pad(ei_assignment[0])] +
        [bc(pad(ea_assignment[:, k])) for k in range(3)], 4, KG)
    str_c = _pack_stream(
        [pad(ei_completion[0]), bc(pad(ea_completion[:, 0]))], 2, KG)
    str_l = _pack_stream(
        [pad(ei_logical[1], NO), pad(ei_logical[0], NO)], 2, KS)
    str_t = _pack_stream(
        [pad(ei_type_valid[1], NM), pad(ei_type_valid[0], NM)], 2, KS)

    mesh = plsc.VectorSubcoreMesh(core_axis_name="c", subcore_axis_name="s")
    sc = pl.kernel(
        _sc_body,
        out_type=(jax.ShapeDtypeStruct((32, H), f32),
                  jax.ShapeDtypeStruct((32, H), f32),
                  jax.ShapeDtypeStruct((NL_TAB,), f32),
                  jax.ShapeDtypeStruct((NT_TAB,), f32),
                  jax.ShapeDtypeStruct((2 * NS * NL_TAB,), f32),
                  jax.ShapeDtypeStruct((2 * NL_TAB,), f32)),
        mesh=mesh,
        compiler_params=pltpu.CompilerParams(use_tc_tiling_on_sc=False,
                                             needs_layout_passes=False),
        scratch_types=[
            pltpu.VMEM((NL_TAB // 2,), f32),
            pltpu.VMEM((NL_TAB,), f32),
            pltpu.VMEM((2 * KS,), i32),
            pltpu.VMEM((2 * KS,), i32),
            pltpu.VMEM((KG, H), f32),
            pltpu.VMEM((KG, H), f32),
            pltpu.VMEM((3, H), f32),
            pltpu.VMEM((H,), f32),
            pltpu.VMEM((H,), f32),
            pltpu.VMEM((H,), f32),
            pltpu.VMEM((H,), f32),
            pltpu.SemaphoreType.DMA,
            pltpu.SemaphoreType.DMA,
            pltpu.SemaphoreType.DMA,
            pltpu.SemaphoreType.DMA,
        ],
    )
    saP, scP, qL, qT, _, _ = sc(xm, xo, str_a, str_c, str_l, str_t,
                                W_ea, b_ea, W_ec[0], b_ec)

    out = pl.pallas_call(
        _combine_body,
        out_shape=jax.ShapeDtypeStruct((1, 2 * H), f32),
    )(saP, scP, qL, qT, x_machine.T, x_operation.T,
      W_me, b_me, W_oe, b_oe, b_ea, b_ec,
      W_ga, b_ga, W_gc, b_gc, W_ll, b_ll, W_lr, W_tl, b_tl, W_tr)
    return out
